# Initial kernel scaffold; baseline (speedup 1.0000x reference)
import jax
import jax.numpy as jnp
from jax.experimental import pallas as pl


def kernel(features, type_mask, edge_index0, mp_idx0, edge_index1, mp_idx1, attn0, attn1, fc1_w, fc1_b, fc2_w, fc_w, fc_b):
    raise NotImplementedError("write your pallas kernel here")


# speedup vs baseline: 2.5410x; 2.5410x over previous
import jax
import jax.numpy as jnp
from jax import lax
from jax.experimental import pallas as pl
from jax.experimental.pallas import tpu as pltpu
from jax.experimental.pallas import tpu_sc as plsc

_N = 10000
_E = 160000
_D = 128
_H = 4
_AV = 128
_OUT = 128

_NC = 2
_NS = 16
_NW = _NC * _NS
_L = 16

_EPT1 = _E // _NW
_B1 = 40
_NB1 = _EPT1 // _B1

_EPT2 = _E // _NS
_B2 = 80
_NB2 = _EPT2 // _B2

_RPS = _N // _NS
_ZR = 25

_mesh = plsc.VectorSubcoreMesh(
    core_axis_name="c", subcore_axis_name="s",
    num_cores=_NC, num_subcores=_NS)


def _k1_body(feat, ptbl, i0, i1, i2, hid_out, ea_out,
             rA, rB, rC, pA, pB, pC, iA, iB, iC, sem):
    c = lax.axis_index("c")
    s = lax.axis_index("s")
    wid = s * _NC + c
    tbase = wid * _EPT1
    third = jnp.float32(1.0 / 3.0)

    def blk(j, _):
        base = tbase + j * _B1
        pltpu.sync_copy(i0.at[pl.ds(base, _B1)], iA)
        pltpu.sync_copy(i1.at[pl.ds(base, _B1)], iB)
        pltpu.sync_copy(i2.at[pl.ds(base, _B1)], iC)
        ca = pltpu.async_copy(feat.at[iA], rA, sem)
        cb = pltpu.async_copy(feat.at[iB], rB, sem)
        cc = pltpu.async_copy(feat.at[iC], rC, sem)
        da = pltpu.async_copy(ptbl.at[iA], pA, sem)
        db = pltpu.async_copy(ptbl.at[iB], pB, sem)
        dc = pltpu.async_copy(ptbl.at[iC], pC, sem)
        ca.wait(); cb.wait(); cc.wait(); da.wait(); db.wait(); dc.wait()

        def edge(b, _):
            for q in range(_D // _L):
                sl = pl.ds(q * _L, _L)
                rA[b, sl] = (rA[b, sl] + rB[b, sl] + rC[b, sl]) * third
            a = (pA[b, pl.ds(0, _L)] + pB[b, pl.ds(0, _L)]
                 + pC[b, pl.ds(0, _L)]) * third
            a = jnp.where(a > 0, a, a * jnp.float32(0.01))
            pA[b, pl.ds(0, _L)] = jnp.exp(a)
            return 0

        lax.fori_loop(0, _B1, edge, 0)
        pltpu.sync_copy(rA, hid_out.at[pl.ds(base, _B1)])
        pltpu.sync_copy(pA, ea_out.at[pl.ds(base, _B1)])
        return 0

    lax.fori_loop(0, _NB1, blk, 0)


_k1 = pl.kernel(
    _k1_body,
    out_type=(jax.ShapeDtypeStruct((_E, _D), jnp.float32),
              jax.ShapeDtypeStruct((_E, _L), jnp.float32)),
    mesh=_mesh,
    compiler_params=pltpu.CompilerParams(use_tc_tiling_on_sc=False, needs_layout_passes=False),
    scratch_types=[
        pltpu.VMEM((_B1, _D), jnp.float32),
        pltpu.VMEM((_B1, _D), jnp.float32),
        pltpu.VMEM((_B1, _D), jnp.float32),
        pltpu.VMEM((_B1, _L), jnp.float32),
        pltpu.VMEM((_B1, _L), jnp.float32),
        pltpu.VMEM((_B1, _L), jnp.float32),
        pltpu.VMEM((_B1,), jnp.int32),
        pltpu.VMEM((_B1,), jnp.int32),
        pltpu.VMEM((_B1,), jnp.int32),
        pltpu.SemaphoreType.DMA,
    ],
)


def _k2_body(hid0, ea0, dst0, hid1, ea1, dst1, U_out, dn_out,
             spU, spD, hb, eb, mb, db, ib, zb, zd):
    c = lax.axis_index("c")
    s = lax.axis_index("s")
    zvec = jnp.zeros((_L,), jnp.float32)

    def z1(i, _):
        zb[i // (_D // _L), pl.ds((i % (_D // _L)) * _L, _L)] = zvec
        return 0
    lax.fori_loop(0, _ZR * (_D // _L), z1, 0)

    def z2(i, _):
        zd[i, pl.ds(0, _L)] = zvec
        return 0
    lax.fori_loop(0, _ZR, z2, 0)

    rbase = s * _RPS
    for m in range(2):
        hid, ea, dst = ((hid0, ea0, dst0), (hid1, ea1, dst1))[m]
        for p in range(2):
            for k in range(_RPS // _ZR):
                pltpu.sync_copy(zb, spU.at[pl.ds(rbase + k * _ZR, _ZR)])
                pltpu.sync_copy(zd, spD.at[pl.ds(rbase + k * _ZR, _ZR)])
            plsc.subcore_barrier()

            lane = jnp.int32(4 * m + 2 * p) + c
            tbase = s * _EPT2

            def blk(j, _):
                base = tbase + j * _B2
                pltpu.sync_copy(hid.at[pl.ds(base, _B2)], hb)
                pltpu.sync_copy(ea.at[pl.ds(base, _B2)], eb)
                pltpu.sync_copy(dst.at[pl.ds(base, _B2)], ib)

                def edge(b, _):
                    wv = plsc.load_gather(
                        eb, (jnp.full((_L,), b, jnp.int32),
                             jnp.full((_L,), lane, jnp.int32)))
                    for q in range(_D // _L):
                        sl = pl.ds(q * _L, _L)
                        mb[b, sl] = hb[b, sl] * wv
                    db[b, pl.ds(0, _L)] = wv
                    return 0

                lax.fori_loop(0, _B2, edge, 0)
                pltpu.sync_copy(mb, spU.at[ib], add=True)
                pltpu.sync_copy(db, spD.at[ib], add=True)
                return 0

            lax.fori_loop(0, _NB2, blk, 0)
            plsc.subcore_barrier()

            h0 = 2 * p

            @pl.when(c == 0)
            def _():
                pltpu.sync_copy(spU.at[pl.ds(rbase, _RPS)],
                                U_out.at[m, h0, pl.ds(rbase, _RPS)])
                pltpu.sync_copy(spD.at[pl.ds(rbase, _RPS)],
                                dn_out.at[m, h0, pl.ds(rbase, _RPS)])

            @pl.when(c == 1)
            def _():
                pltpu.sync_copy(spU.at[pl.ds(rbase, _RPS)],
                                U_out.at[m, h0 + 1, pl.ds(rbase, _RPS)])
                pltpu.sync_copy(spD.at[pl.ds(rbase, _RPS)],
                                dn_out.at[m, h0 + 1, pl.ds(rbase, _RPS)])


_k2 = pl.kernel(
    _k2_body,
    out_type=(jax.ShapeDtypeStruct((2, _H, _N, _D), jnp.float32),
              jax.ShapeDtypeStruct((2, _H, _N, _L), jnp.float32)),
    mesh=_mesh,
    compiler_params=pltpu.CompilerParams(use_tc_tiling_on_sc=False, needs_layout_passes=False),
    scratch_types=[
        pltpu.VMEM_SHARED((_N, _D), jnp.float32),
        pltpu.VMEM_SHARED((_N, _L), jnp.float32),
        pltpu.VMEM((_B2, _D), jnp.float32),
        pltpu.VMEM((_B2, _L), jnp.float32),
        pltpu.VMEM((_B2, _D), jnp.float32),
        pltpu.VMEM((_B2, _L), jnp.float32),
        pltpu.VMEM((_B2,), jnp.int32),
        pltpu.VMEM((_ZR, _D), jnp.float32),
        pltpu.VMEM((_ZR, _L), jnp.float32),
    ],
)


_BN = 1000


def _t0_body(f_ref, w_ref, o_ref):
    o_ref[...] = jnp.dot(f_ref[...], w_ref[...],
                         preferred_element_type=jnp.float32)


def _elu_rows(u_ref, d_ref, m, h):
    u = u_ref[m, h]
    d = d_ref[m, h, :, 0:1]
    r = jnp.where(d > 0, u / d, 0.0)
    return jnp.where(r > 0, r, jnp.exp(r) - 1.0)


def _t1_body(u_ref, d_ref, w_ref, b_ref, o_ref):
    i = pl.program_id(1)

    @pl.when(i == 0)
    def _():
        o_ref[...] = jnp.zeros_like(o_ref)

    acc = jnp.broadcast_to(b_ref[...], (_BN, _AV))
    for h in range(_H):
        o = _elu_rows(u_ref, d_ref, 0, h)
        acc = acc + jnp.dot(o, w_ref[h], preferred_element_type=jnp.float32)
    col = jnp.sum(jnp.tanh(acc), axis=0, keepdims=True)
    o_ref[...] += jnp.broadcast_to(col, (1, 8, _AV))


def _t2_body(u_ref, d_ref, tm_ref, beta_ref, w2_ref, b2_ref, h_ref, hfc_ref):
    mask = tm_ref[...] == 0
    b0 = beta_ref[0]
    b1 = beta_ref[1]
    acc = jnp.broadcast_to(b2_ref[...], (_BN, _OUT))
    for h in range(_H):
        hb = b0 * _elu_rows(u_ref, d_ref, 0, h) + b1 * _elu_rows(u_ref, d_ref, 1, h)
        hb = jnp.where(mask, hb, 0.0)
        h_ref[h] = hb
        acc = acc + jnp.dot(hb, w2_ref[h], preferred_element_type=jnp.float32)
    hfc_ref[...] = acc


def kernel(features, type_mask, edge_index0, mp_idx0, edge_index1, mp_idx1,
           attn0, attn1, fc1_w, fc1_b, fc2_w, fc_w, fc_b):
    f32 = jnp.float32
    wp = jnp.concatenate(
        [attn0[0].T, attn1[0].T, jnp.zeros((_D, 2 * _H), f32)], axis=1)
    mpT0 = mp_idx0.T
    mpT1 = mp_idx1.T
    dst0 = edge_index0[1]
    dst1 = edge_index1[1]
    w1r = fc1_w.reshape(_AV, _H, _D).transpose(1, 2, 0)
    w2r = fc_w.reshape(_OUT, _H, _D).transpose(1, 2, 0)

    ptbl = pl.pallas_call(
        _t0_body,
        grid=(_N // _BN,),
        in_specs=[pl.BlockSpec((_BN, _D), lambda i: (i, 0)),
                  pl.BlockSpec((_D, _L), lambda i: (0, 0))],
        out_specs=pl.BlockSpec((_BN, _L), lambda i: (i, 0)),
        out_shape=jax.ShapeDtypeStruct((_N, _L), f32),
    )(features, wp)

    hid0, ea0 = _k1(features, ptbl, mpT0[0], mpT0[1], mpT0[2])
    hid1, ea1 = _k1(features, ptbl, mpT1[0], mpT1[1], mpT1[2])

    U, dn = _k2(hid0, ea0, dst0, hid1, ea1, dst1)

    colsum = pl.pallas_call(
        _t1_body,
        grid=(2, _N // _BN),
        in_specs=[
            pl.BlockSpec((1, _H, _BN, _D), lambda m, i: (m, 0, i, 0)),
            pl.BlockSpec((1, _H, _BN, _L), lambda m, i: (m, 0, i, 0)),
            pl.BlockSpec((_H, _D, _AV), lambda m, i: (0, 0, 0)),
            pl.BlockSpec((1, _AV), lambda m, i: (0, 0)),
        ],
        out_specs=pl.BlockSpec((1, 8, _AV), lambda m, i: (m, 0, 0)),
        out_shape=jax.ShapeDtypeStruct((2, 8, _AV), f32),
    )(U, dn, w1r, fc1_b.reshape(1, _AV))

    s_mean = colsum[:, 0, :] / _N
    beta = jax.nn.softmax((s_mean @ fc2_w.T)[:, 0])

    h_hnd, h_fc = pl.pallas_call(
        _t2_body,
        grid=(_N // _BN,),
        in_specs=[
            pl.BlockSpec((2, _H, _BN, _D), lambda i: (0, 0, i, 0)),
            pl.BlockSpec((2, _H, _BN, _L), lambda i: (0, 0, i, 0)),
            pl.BlockSpec((_BN, 1), lambda i: (i, 0)),
            pl.BlockSpec(memory_space=pltpu.SMEM),
            pl.BlockSpec((_H, _D, _OUT), lambda i: (0, 0, 0)),
            pl.BlockSpec((1, _OUT), lambda i: (0, 0)),
        ],
        out_specs=[
            pl.BlockSpec((_H, _BN, _D), lambda i: (0, i, 0)),
            pl.BlockSpec((_BN, _OUT), lambda i: (i, 0)),
        ],
        out_shape=[
            jax.ShapeDtypeStruct((_H, _N, _D), f32),
            jax.ShapeDtypeStruct((_N, _OUT), f32),
        ],
    )(U, dn, type_mask.reshape(_N, 1), beta, w2r, fc_b.reshape(1, _OUT))

    h = h_hnd.transpose(1, 0, 2).reshape(_N, _H * _D)
    return h_fc, h, beta[:, None, None]


# speedup vs baseline: 4.2533x; 1.6739x over previous
import jax
import jax.numpy as jnp
from jax import lax
from jax.experimental import pallas as pl
from jax.experimental.pallas import tpu as pltpu
from jax.experimental.pallas import tpu_sc as plsc

_N = 10000
_E = 160000
_D = 128
_H = 4
_AV = 128
_OUT = 128

_NC = 2
_NS = 16
_NW = _NC * _NS
_L = 16

_EPT1 = _E // _NW
_B1 = 40
_NB1 = _EPT1 // _B1

_EPT2 = _E // _NS
_B2 = 40
_NB2 = _EPT2 // _B2

_RPS = _N // _NS
_ZR = 25

_mesh = plsc.VectorSubcoreMesh(
    core_axis_name="c", subcore_axis_name="s",
    num_cores=_NC, num_subcores=_NS)


def _k1_body(feat, ptbl, mpT, hid_out, ea_out,
             rA0, rB0, rC0, pA0, pB0, pC0, ix0,
             rA1, rB1, rC1, pA1, pB1, pC1, ix1, sem0, sem1):
    c = lax.axis_index("c")
    s = lax.axis_index("s")
    wid = s * _NC + c
    tbase = wid * _EPT1
    third = jnp.float32(1.0 / 3.0)

    sets = ((rA0, rB0, rC0, pA0, pB0, pC0, ix0, sem0),
            (rA1, rB1, rC1, pA1, pB1, pC1, ix1, sem1))

    def fire(P, j):
        rA, rB, rC, pA, pB, pC, ix, sem = sets[P]
        base = tbase + j * _B1
        pltpu.sync_copy(mpT.at[:, pl.ds(base, _B1)], ix)
        pltpu.async_copy(feat.at[ix.at[0]], rA, sem)
        pltpu.async_copy(feat.at[ix.at[1]], rB, sem)
        pltpu.async_copy(feat.at[ix.at[2]], rC, sem)
        pltpu.async_copy(ptbl.at[ix.at[0]], pA, sem)
        pltpu.async_copy(ptbl.at[ix.at[1]], pB, sem)
        pltpu.async_copy(ptbl.at[ix.at[2]], pC, sem)

    def waitset(P):
        rA, rB, rC, pA, pB, pC, ix, sem = sets[P]
        pltpu.make_async_copy(feat.at[ix.at[0]], rA, sem).wait()
        pltpu.make_async_copy(feat.at[ix.at[1]], rB, sem).wait()
        pltpu.make_async_copy(feat.at[ix.at[2]], rC, sem).wait()
        pltpu.make_async_copy(ptbl.at[ix.at[0]], pA, sem).wait()
        pltpu.make_async_copy(ptbl.at[ix.at[1]], pB, sem).wait()
        pltpu.make_async_copy(ptbl.at[ix.at[2]], pC, sem).wait()

    def compute(P, j):
        rA, rB, rC, pA, pB, pC, ix, sem = sets[P]
        base = tbase + j * _B1

        def edge4(t, _):
            for u in range(4):
                b = t * 4 + u
                for q in range(_D // _L):
                    sl = pl.ds(q * _L, _L)
                    rA[b, sl] = (rA[b, sl] + rB[b, sl] + rC[b, sl]) * third
                a = (pA[b, pl.ds(0, _L)] + pB[b, pl.ds(0, _L)]
                     + pC[b, pl.ds(0, _L)]) * third
                a = jnp.where(a > 0, a, a * jnp.float32(0.01))
                pA[b, pl.ds(0, _L)] = jnp.exp(a)
            return 0

        lax.fori_loop(0, _B1 // 4, edge4, 0)
        pltpu.sync_copy(rA, hid_out.at[pl.ds(base, _B1)])
        pltpu.sync_copy(pA, ea_out.at[pl.ds(base, _B1)])

    fire(0, 0)

    def pair(k, _):
        fire(1, 2 * k + 1)
        waitset(0)
        compute(0, 2 * k)
        fire(0, 2 * k + 2)
        waitset(1)
        compute(1, 2 * k + 1)
        return 0

    lax.fori_loop(0, (_NB1 - 1) // 2, pair, 0)
    waitset(0)
    compute(0, _NB1 - 1)


_k1 = pl.kernel(
    _k1_body,
    out_type=(jax.ShapeDtypeStruct((_E, _D), jnp.float32),
              jax.ShapeDtypeStruct((_E, _L), jnp.float32)),
    mesh=_mesh,
    compiler_params=pltpu.CompilerParams(use_tc_tiling_on_sc=False, needs_layout_passes=False),
    scratch_types=[
        pltpu.VMEM((_B1, _D), jnp.float32),
        pltpu.VMEM((_B1, _D), jnp.float32),
        pltpu.VMEM((_B1, _D), jnp.float32),
        pltpu.VMEM((_B1, _L), jnp.float32),
        pltpu.VMEM((_B1, _L), jnp.float32),
        pltpu.VMEM((_B1, _L), jnp.float32),
        pltpu.VMEM((3, _B1), jnp.int32),
        pltpu.VMEM((_B1, _D), jnp.float32),
        pltpu.VMEM((_B1, _D), jnp.float32),
        pltpu.VMEM((_B1, _D), jnp.float32),
        pltpu.VMEM((_B1, _L), jnp.float32),
        pltpu.VMEM((_B1, _L), jnp.float32),
        pltpu.VMEM((_B1, _L), jnp.float32),
        pltpu.VMEM((3, _B1), jnp.int32),
        pltpu.SemaphoreType.DMA,
        pltpu.SemaphoreType.DMA,
    ],
)


def _k2_body(hid0, ea0, dst0, hid1, ea1, dst1, U_out, dn_out,
             spU, spD,
             hb0, eb0, mb0, db0, ib0, is0,
             hb1, eb1, mb1, db1, ib1, is1,
             zb, zd, sin0, ssc0, sin1, ssc1):
    c = lax.axis_index("c")
    s = lax.axis_index("s")
    zvec = jnp.zeros((_L,), jnp.float32)

    def z1(i, _):
        zb[i // (_D // _L), pl.ds((i % (_D // _L)) * _L, _L)] = zvec
        return 0
    lax.fori_loop(0, _ZR * (_D // _L), z1, 0)

    def z2(i, _):
        zd[i, pl.ds(0, _L)] = zvec
        return 0
    lax.fori_loop(0, _ZR, z2, 0)

    sets = ((hb0, eb0, mb0, db0, ib0, is0, sin0, ssc0),
            (hb1, eb1, mb1, db1, ib1, is1, sin1, ssc1))
    rbase = s * _RPS
    for m in range(2):
        hid, ea, dst = ((hid0, ea0, dst0), (hid1, ea1, dst1))[m]
        for p in range(2):
            for k in range(_RPS // _ZR):
                pltpu.sync_copy(zb, spU.at[pl.ds(rbase + k * _ZR, _ZR)])
                pltpu.sync_copy(zd, spD.at[pl.ds(rbase + k * _ZR, _ZR)])
            plsc.subcore_barrier()

            lane = jnp.int32(4 * m + 2 * p) + c
            tbase = s * _EPT2

            def fire_in(P, j):
                hb, eb, mb, db, ib, isc, sin, ssc = sets[P]
                base = tbase + j * _B2
                pltpu.async_copy(hid.at[pl.ds(base, _B2)], hb, sin)
                pltpu.async_copy(ea.at[pl.ds(base, _B2)], eb, sin)
                pltpu.async_copy(dst.at[pl.ds(base, _B2)], ib, sin)

            def wait_in(P):
                hb, eb, mb, db, ib, isc, sin, ssc = sets[P]
                pltpu.make_async_copy(hid.at[pl.ds(tbase, _B2)], hb, sin).wait()
                pltpu.make_async_copy(ea.at[pl.ds(tbase, _B2)], eb, sin).wait()
                pltpu.make_async_copy(dst.at[pl.ds(tbase, _B2)], ib, sin).wait()

            def wait_sc(P):
                hb, eb, mb, db, ib, isc, sin, ssc = sets[P]
                pltpu.make_async_copy(mb, spU.at[isc], ssc).wait()
                pltpu.make_async_copy(db, spD.at[isc], ssc).wait()

            def compute_fire(P, first):
                hb, eb, mb, db, ib, isc, sin, ssc = sets[P]
                starts = list(range(0, _B2 - _L + 1, _L))
                if starts[-1] != _B2 - _L:
                    starts.append(_B2 - _L)
                for st in starts:
                    sl = pl.ds(st, _L)
                    isc[sl] = ib[sl]

                def edge4(t, _):
                    for u in range(4):
                        b = t * 4 + u
                        wv = plsc.load_gather(
                            eb, (jnp.full((_L,), b, jnp.int32),
                                 jnp.full((_L,), lane, jnp.int32)))
                        for q in range(_D // _L):
                            sl = pl.ds(q * _L, _L)
                            mb[b, sl] = hb[b, sl] * wv
                        db[b, pl.ds(0, _L)] = wv
                    return 0

                lax.fori_loop(0, _B2 // 4, edge4, 0)
                pltpu.async_copy(mb, spU.at[isc], ssc, add=True)
                pltpu.async_copy(db, spD.at[isc], ssc, add=True)

            fire_in(0, 0)

            def blkpair(k, _):
                fire_in(1, 2 * k + 1)
                wait_in(0)

                @pl.when(k > 0)
                def _():
                    wait_sc(0)
                compute_fire(0, k == 0)

                @pl.when(k < _NB2 // 2 - 1)
                def _():
                    fire_in(0, 2 * k + 2)
                wait_in(1)

                @pl.when(k > 0)
                def _():
                    wait_sc(1)
                compute_fire(1, k == 0)
                return 0

            lax.fori_loop(0, _NB2 // 2, blkpair, 0)
            wait_sc(0)
            wait_sc(1)
            plsc.subcore_barrier()

            h0 = 2 * p

            @pl.when(c == 0)
            def _():
                pltpu.sync_copy(spU.at[pl.ds(rbase, _RPS)],
                                U_out.at[m, h0, pl.ds(rbase, _RPS)])
                pltpu.sync_copy(spD.at[pl.ds(rbase, _RPS)],
                                dn_out.at[m, h0, pl.ds(rbase, _RPS)])

            @pl.when(c == 1)
            def _():
                pltpu.sync_copy(spU.at[pl.ds(rbase, _RPS)],
                                U_out.at[m, h0 + 1, pl.ds(rbase, _RPS)])
                pltpu.sync_copy(spD.at[pl.ds(rbase, _RPS)],
                                dn_out.at[m, h0 + 1, pl.ds(rbase, _RPS)])


_k2 = pl.kernel(
    _k2_body,
    out_type=(jax.ShapeDtypeStruct((2, _H, _N, _D), jnp.float32),
              jax.ShapeDtypeStruct((2, _H, _N, _L), jnp.float32)),
    mesh=_mesh,
    compiler_params=pltpu.CompilerParams(use_tc_tiling_on_sc=False, needs_layout_passes=False),
    scratch_types=[
        pltpu.VMEM_SHARED((_N, _D), jnp.float32),
        pltpu.VMEM_SHARED((_N, _L), jnp.float32),
        pltpu.VMEM((_B2, _D), jnp.float32),
        pltpu.VMEM((_B2, _L), jnp.float32),
        pltpu.VMEM((_B2, _D), jnp.float32),
        pltpu.VMEM((_B2, _L), jnp.float32),
        pltpu.VMEM((_B2,), jnp.int32),
        pltpu.VMEM((_B2,), jnp.int32),
        pltpu.VMEM((_B2, _D), jnp.float32),
        pltpu.VMEM((_B2, _L), jnp.float32),
        pltpu.VMEM((_B2, _D), jnp.float32),
        pltpu.VMEM((_B2, _L), jnp.float32),
        pltpu.VMEM((_B2,), jnp.int32),
        pltpu.VMEM((_B2,), jnp.int32),
        pltpu.VMEM((_ZR, _D), jnp.float32),
        pltpu.VMEM((_ZR, _L), jnp.float32),
        pltpu.SemaphoreType.DMA,
        pltpu.SemaphoreType.DMA,
        pltpu.SemaphoreType.DMA,
        pltpu.SemaphoreType.DMA,
    ],
)


_BN = 1000


def _t0_body(f_ref, w_ref, o_ref):
    o_ref[...] = jnp.dot(f_ref[...], w_ref[...],
                         preferred_element_type=jnp.float32)


def _elu_rows(u_ref, d_ref, m, h):
    u = u_ref[m, h]
    d = d_ref[m, h, :, 0:1]
    r = jnp.where(d > 0, u / d, 0.0)
    return jnp.where(r > 0, r, jnp.exp(r) - 1.0)


def _t1_body(u_ref, d_ref, w_ref, b_ref, o_ref):
    i = pl.program_id(1)

    @pl.when(i == 0)
    def _():
        o_ref[...] = jnp.zeros_like(o_ref)

    acc = jnp.broadcast_to(b_ref[...], (_BN, _AV))
    for h in range(_H):
        o = _elu_rows(u_ref, d_ref, 0, h)
        acc = acc + jnp.dot(o, w_ref[h], preferred_element_type=jnp.float32)
    col = jnp.sum(jnp.tanh(acc), axis=0, keepdims=True)
    o_ref[...] += jnp.broadcast_to(col, (1, 8, _AV))


def _t2_body(u_ref, d_ref, tm_ref, beta_ref, w2_ref, b2_ref, h_ref, hfc_ref):
    mask = tm_ref[...] == 0
    b0 = beta_ref[0]
    b1 = beta_ref[1]
    acc = jnp.broadcast_to(b2_ref[...], (_BN, _OUT))
    for h in range(_H):
        hb = b0 * _elu_rows(u_ref, d_ref, 0, h) + b1 * _elu_rows(u_ref, d_ref, 1, h)
        hb = jnp.where(mask, hb, 0.0)
        h_ref[h] = hb
        acc = acc + jnp.dot(hb, w2_ref[h], preferred_element_type=jnp.float32)
    hfc_ref[...] = acc


def kernel(features, type_mask, edge_index0, mp_idx0, edge_index1, mp_idx1,
           attn0, attn1, fc1_w, fc1_b, fc2_w, fc_w, fc_b):
    f32 = jnp.float32
    wp = jnp.concatenate(
        [attn0[0].T, attn1[0].T, jnp.zeros((_D, 2 * _H), f32)], axis=1)
    mpT0 = mp_idx0.T
    mpT1 = mp_idx1.T
    dst0 = edge_index0[1]
    dst1 = edge_index1[1]
    w1r = fc1_w.reshape(_AV, _H, _D).transpose(1, 2, 0)
    w2r = fc_w.reshape(_OUT, _H, _D).transpose(1, 2, 0)

    ptbl = pl.pallas_call(
        _t0_body,
        grid=(_N // _BN,),
        in_specs=[pl.BlockSpec((_BN, _D), lambda i: (i, 0)),
                  pl.BlockSpec((_D, _L), lambda i: (0, 0))],
        out_specs=pl.BlockSpec((_BN, _L), lambda i: (i, 0)),
        out_shape=jax.ShapeDtypeStruct((_N, _L), f32),
    )(features, wp)

    hid0, ea0 = _k1(features, ptbl, mpT0)
    hid1, ea1 = _k1(features, ptbl, mpT1)

    U, dn = _k2(hid0, ea0, dst0, hid1, ea1, dst1)

    colsum = pl.pallas_call(
        _t1_body,
        grid=(2, _N // _BN),
        in_specs=[
            pl.BlockSpec((1, _H, _BN, _D), lambda m, i: (m, 0, i, 0)),
            pl.BlockSpec((1, _H, _BN, _L), lambda m, i: (m, 0, i, 0)),
            pl.BlockSpec((_H, _D, _AV), lambda m, i: (0, 0, 0)),
            pl.BlockSpec((1, _AV), lambda m, i: (0, 0)),
        ],
        out_specs=pl.BlockSpec((1, 8, _AV), lambda m, i: (m, 0, 0)),
        out_shape=jax.ShapeDtypeStruct((2, 8, _AV), f32),
    )(U, dn, w1r, fc1_b.reshape(1, _AV))

    s_mean = colsum[:, 0, :] / _N
    beta = jax.nn.softmax((s_mean @ fc2_w.T)[:, 0])

    h_hnd, h_fc = pl.pallas_call(
        _t2_body,
        grid=(_N // _BN,),
        in_specs=[
            pl.BlockSpec((2, _H, _BN, _D), lambda i: (0, 0, i, 0)),
            pl.BlockSpec((2, _H, _BN, _L), lambda i: (0, 0, i, 0)),
            pl.BlockSpec((_BN, 1), lambda i: (i, 0)),
            pl.BlockSpec(memory_space=pltpu.SMEM),
            pl.BlockSpec((_H, _D, _OUT), lambda i: (0, 0, 0)),
            pl.BlockSpec((1, _OUT), lambda i: (0, 0)),
        ],
        out_specs=[
            pl.BlockSpec((_H, _BN, _D), lambda i: (0, i, 0)),
            pl.BlockSpec((_BN, _OUT), lambda i: (i, 0)),
        ],
        out_shape=[
            jax.ShapeDtypeStruct((_H, _N, _D), f32),
            jax.ShapeDtypeStruct((_N, _OUT), f32),
        ],
    )(U, dn, type_mask.reshape(_N, 1), beta, w2r, fc_b.reshape(1, _OUT))

    h = h_hnd.transpose(1, 0, 2).reshape(_N, _H * _D)
    return h_fc, h, beta[:, None, None]


# speedup vs baseline: 9.3061x; 2.1880x over previous
import jax
import jax.numpy as jnp
from jax import lax
from jax.experimental import pallas as pl
from jax.experimental.pallas import tpu as pltpu
from jax.experimental.pallas import tpu_sc as plsc

_N = 10000
_E = 160000
_D = 128
_H = 4
_AV = 128
_OUT = 128

_NC = 2
_NS = 16
_NW = _NC * _NS
_L = 16

_EPT1 = _E // _NW
_B1 = 40
_NB1 = _EPT1 // _B1

_EPT2 = _E // _NS
_B2 = 40
_NB2 = _EPT2 // _B2

_RPS = _N // _NS
_ZR = 25

_mesh = plsc.VectorSubcoreMesh(
    core_axis_name="c", subcore_axis_name="s",
    num_cores=_NC, num_subcores=_NS)


def _k1_body(ftbl, mpT, hid_out, ea_out,
             rA0, rB0, rC0, hB0, eB0, ix0,
             rA1, rB1, rC1, hB1, eB1, ix1, sem0, sem1):
    c = lax.axis_index("c")
    s = lax.axis_index("s")
    wid = s * _NC + c
    tbase = wid * _EPT1
    third = jnp.float32(1.0 / 3.0)

    sets = ((rA0, rB0, rC0, hB0, eB0, ix0, sem0),
            (rA1, rB1, rC1, hB1, eB1, ix1, sem1))

    def fire(P, j):
        rA, rB, rC, hB, eB, ix, sem = sets[P]
        base = tbase + j * _B1
        pltpu.sync_copy(mpT.at[:, pl.ds(base, _B1)], ix)
        pltpu.async_copy(ftbl.at[ix.at[0]], rA, sem)
        pltpu.async_copy(ftbl.at[ix.at[1]], rB, sem)
        pltpu.async_copy(ftbl.at[ix.at[2]], rC, sem)

    def waitset(P):
        rA, rB, rC, hB, eB, ix, sem = sets[P]
        pltpu.make_async_copy(ftbl.at[ix.at[0]], rA, sem).wait()
        pltpu.make_async_copy(ftbl.at[ix.at[1]], rB, sem).wait()
        pltpu.make_async_copy(ftbl.at[ix.at[2]], rC, sem).wait()

    def compute(P, j):
        rA, rB, rC, hB, eB, ix, sem = sets[P]
        base = tbase + j * _B1

        @plsc.parallel_loop(0, _B1, step=1, unroll=4)
        def _(b):
            for q in range(_D // _L):
                sl = pl.ds(q * _L, _L)
                hB[b, sl] = (rA[b, sl] + rB[b, sl] + rC[b, sl]) * third
            pp = pl.ds(_D, _L)
            a = (rA[b, pp] + rB[b, pp] + rC[b, pp]) * third
            a = jnp.where(a > 0, a, a * jnp.float32(0.01))
            eB[b, pl.ds(0, _L)] = jnp.exp(a)

        pltpu.sync_copy(hB, hid_out.at[pl.ds(base, _B1)])
        pltpu.sync_copy(eB, ea_out.at[pl.ds(base, _B1)])

    fire(0, 0)

    def pair(k, _):
        fire(1, 2 * k + 1)
        waitset(0)
        compute(0, 2 * k)
        fire(0, 2 * k + 2)
        waitset(1)
        compute(1, 2 * k + 1)
        return 0

    lax.fori_loop(0, (_NB1 - 1) // 2, pair, 0)
    waitset(0)
    compute(0, _NB1 - 1)


_k1 = pl.kernel(
    _k1_body,
    out_type=(jax.ShapeDtypeStruct((_E, _D), jnp.float32),
              jax.ShapeDtypeStruct((_E, _L), jnp.float32)),
    mesh=_mesh,
    compiler_params=pltpu.CompilerParams(use_tc_tiling_on_sc=False, needs_layout_passes=False),
    scratch_types=[
        pltpu.VMEM((_B1, _D + _L), jnp.float32),
        pltpu.VMEM((_B1, _D + _L), jnp.float32),
        pltpu.VMEM((_B1, _D + _L), jnp.float32),
        pltpu.VMEM((_B1, _D), jnp.float32),
        pltpu.VMEM((_B1, _L), jnp.float32),
        pltpu.VMEM((3, _B1), jnp.int32),
        pltpu.VMEM((_B1, _D + _L), jnp.float32),
        pltpu.VMEM((_B1, _D + _L), jnp.float32),
        pltpu.VMEM((_B1, _D + _L), jnp.float32),
        pltpu.VMEM((_B1, _D), jnp.float32),
        pltpu.VMEM((_B1, _L), jnp.float32),
        pltpu.VMEM((3, _B1), jnp.int32),
        pltpu.SemaphoreType.DMA,
        pltpu.SemaphoreType.DMA,
    ],
)


def _k2_body(hid0, ea0, dst0, hid1, ea1, dst1, U_out, dn_out,
             spU, spD,
             hb0, eb0, mb0, db0, ib0, is0,
             hb1, eb1, mb1, db1, ib1, is1,
             zb, zd, sin0, ssc0, sin1, ssc1):
    c = lax.axis_index("c")
    s = lax.axis_index("s")
    zvec = jnp.zeros((_L,), jnp.float32)

    def z1(i, _):
        zb[i // (_D // _L), pl.ds((i % (_D // _L)) * _L, _L)] = zvec
        return 0
    lax.fori_loop(0, _ZR * (_D // _L), z1, 0)

    def z2(i, _):
        zd[i, pl.ds(0, _L)] = zvec
        return 0
    lax.fori_loop(0, _ZR, z2, 0)

    sets = ((hb0, eb0, mb0, db0, ib0, is0, sin0, ssc0),
            (hb1, eb1, mb1, db1, ib1, is1, sin1, ssc1))
    rbase = s * _RPS
    for m in range(2):
        hid, ea, dst = ((hid0, ea0, dst0), (hid1, ea1, dst1))[m]
        for p in range(2):
            for k in range(_RPS // _ZR):
                pltpu.sync_copy(zb, spU.at[pl.ds(rbase + k * _ZR, _ZR)])
                pltpu.sync_copy(zd, spD.at[pl.ds(rbase + k * _ZR, _ZR)])
            plsc.subcore_barrier()

            lane = jnp.int32(4 * m + 2 * p) + c
            tbase = s * _EPT2

            def fire_in(P, j):
                hb, eb, mb, db, ib, isc, sin, ssc = sets[P]
                base = tbase + j * _B2
                pltpu.async_copy(hid.at[pl.ds(base, _B2)], hb, sin)
                pltpu.async_copy(ea.at[pl.ds(base, _B2)], eb, sin)
                pltpu.async_copy(dst.at[pl.ds(base, _B2)], ib, sin)

            def wait_in(P):
                hb, eb, mb, db, ib, isc, sin, ssc = sets[P]
                pltpu.make_async_copy(hid.at[pl.ds(tbase, _B2)], hb, sin).wait()
                pltpu.make_async_copy(ea.at[pl.ds(tbase, _B2)], eb, sin).wait()
                pltpu.make_async_copy(dst.at[pl.ds(tbase, _B2)], ib, sin).wait()

            def wait_sc(P):
                hb, eb, mb, db, ib, isc, sin, ssc = sets[P]
                pltpu.make_async_copy(mb, spU.at[isc], ssc).wait()
                pltpu.make_async_copy(db, spD.at[isc], ssc).wait()

            def compute_fire(P, first):
                hb, eb, mb, db, ib, isc, sin, ssc = sets[P]
                starts = list(range(0, _B2 - _L + 1, _L))
                if starts[-1] != _B2 - _L:
                    starts.append(_B2 - _L)
                for st in starts:
                    sl = pl.ds(st, _L)
                    isc[sl] = ib[sl]

                @plsc.parallel_loop(0, _B2, step=1, unroll=4)
                def _(b):
                    wv = plsc.load_gather(
                        eb, (jnp.full((_L,), b, jnp.int32),
                             jnp.full((_L,), lane, jnp.int32)))
                    for q in range(_D // _L):
                        sl = pl.ds(q * _L, _L)
                        mb[b, sl] = hb[b, sl] * wv
                    db[b, pl.ds(0, _L)] = wv
                pltpu.async_copy(mb, spU.at[isc], ssc, add=True)
                pltpu.async_copy(db, spD.at[isc], ssc, add=True)

            fire_in(0, 0)

            def blkpair(k, _):
                fire_in(1, 2 * k + 1)
                wait_in(0)

                @pl.when(k > 0)
                def _():
                    wait_sc(0)
                compute_fire(0, k == 0)

                @pl.when(k < _NB2 // 2 - 1)
                def _():
                    fire_in(0, 2 * k + 2)
                wait_in(1)

                @pl.when(k > 0)
                def _():
                    wait_sc(1)
                compute_fire(1, k == 0)
                return 0

            lax.fori_loop(0, _NB2 // 2, blkpair, 0)
            wait_sc(0)
            wait_sc(1)
            plsc.subcore_barrier()

            h0 = 2 * p

            @pl.when(c == 0)
            def _():
                pltpu.sync_copy(spU.at[pl.ds(rbase, _RPS)],
                                U_out.at[m, h0, pl.ds(rbase, _RPS)])
                pltpu.sync_copy(spD.at[pl.ds(rbase, _RPS)],
                                dn_out.at[m, h0, pl.ds(rbase, _RPS)])

            @pl.when(c == 1)
            def _():
                pltpu.sync_copy(spU.at[pl.ds(rbase, _RPS)],
                                U_out.at[m, h0 + 1, pl.ds(rbase, _RPS)])
                pltpu.sync_copy(spD.at[pl.ds(rbase, _RPS)],
                                dn_out.at[m, h0 + 1, pl.ds(rbase, _RPS)])


_k2 = pl.kernel(
    _k2_body,
    out_type=(jax.ShapeDtypeStruct((2, _H, _N, _D), jnp.float32),
              jax.ShapeDtypeStruct((2, _H, _N, _L), jnp.float32)),
    mesh=_mesh,
    compiler_params=pltpu.CompilerParams(use_tc_tiling_on_sc=False, needs_layout_passes=False),
    scratch_types=[
        pltpu.VMEM_SHARED((_N, _D), jnp.float32),
        pltpu.VMEM_SHARED((_N, _L), jnp.float32),
        pltpu.VMEM((_B2, _D), jnp.float32),
        pltpu.VMEM((_B2, _L), jnp.float32),
        pltpu.VMEM((_B2, _D), jnp.float32),
        pltpu.VMEM((_B2, _L), jnp.float32),
        pltpu.VMEM((_B2,), jnp.int32),
        pltpu.VMEM((_B2,), jnp.int32),
        pltpu.VMEM((_B2, _D), jnp.float32),
        pltpu.VMEM((_B2, _L), jnp.float32),
        pltpu.VMEM((_B2, _D), jnp.float32),
        pltpu.VMEM((_B2, _L), jnp.float32),
        pltpu.VMEM((_B2,), jnp.int32),
        pltpu.VMEM((_B2,), jnp.int32),
        pltpu.VMEM((_ZR, _D), jnp.float32),
        pltpu.VMEM((_ZR, _L), jnp.float32),
        pltpu.SemaphoreType.DMA,
        pltpu.SemaphoreType.DMA,
        pltpu.SemaphoreType.DMA,
        pltpu.SemaphoreType.DMA,
    ],
)


_BN = 1000


def _t0_body(f_ref, w_ref, o_ref):
    f = f_ref[...]
    o_ref[:, 0:_D] = f
    o_ref[:, _D:_D + _L] = jnp.dot(f, w_ref[...],
                                   preferred_element_type=jnp.float32)


def _elu_rows(u_ref, d_ref, m, h):
    u = u_ref[m, h]
    d = d_ref[m, h, :, 0:1]
    r = jnp.where(d > 0, u / d, 0.0)
    return jnp.where(r > 0, r, jnp.exp(r) - 1.0)


def _t1_body(u_ref, d_ref, w_ref, b_ref, o_ref):
    i = pl.program_id(1)

    @pl.when(i == 0)
    def _():
        o_ref[...] = jnp.zeros_like(o_ref)

    acc = jnp.broadcast_to(b_ref[...], (_BN, _AV))
    for h in range(_H):
        o = _elu_rows(u_ref, d_ref, 0, h)
        acc = acc + jnp.dot(o, w_ref[h], preferred_element_type=jnp.float32)
    col = jnp.sum(jnp.tanh(acc), axis=0, keepdims=True)
    o_ref[...] += jnp.broadcast_to(col, (1, 8, _AV))


def _t2_body(u_ref, d_ref, tm_ref, beta_ref, w2_ref, b2_ref, h_ref, hfc_ref):
    mask = tm_ref[...] == 0
    b0 = beta_ref[0]
    b1 = beta_ref[1]
    acc = jnp.broadcast_to(b2_ref[...], (_BN, _OUT))
    for h in range(_H):
        hb = b0 * _elu_rows(u_ref, d_ref, 0, h) + b1 * _elu_rows(u_ref, d_ref, 1, h)
        hb = jnp.where(mask, hb, 0.0)
        h_ref[h] = hb
        acc = acc + jnp.dot(hb, w2_ref[h], preferred_element_type=jnp.float32)
    hfc_ref[...] = acc


def kernel(features, type_mask, edge_index0, mp_idx0, edge_index1, mp_idx1,
           attn0, attn1, fc1_w, fc1_b, fc2_w, fc_w, fc_b):
    f32 = jnp.float32
    wp = jnp.concatenate(
        [attn0[0].T, attn1[0].T, jnp.zeros((_D, 2 * _H), f32)], axis=1)
    mpT0 = mp_idx0.T
    mpT1 = mp_idx1.T
    dst0 = edge_index0[1]
    dst1 = edge_index1[1]
    w1r = fc1_w.reshape(_AV, _H, _D).transpose(1, 2, 0)
    w2r = fc_w.reshape(_OUT, _H, _D).transpose(1, 2, 0)

    ftbl = pl.pallas_call(
        _t0_body,
        grid=(_N // _BN,),
        in_specs=[pl.BlockSpec((_BN, _D), lambda i: (i, 0)),
                  pl.BlockSpec((_D, _L), lambda i: (0, 0))],
        out_specs=pl.BlockSpec((_BN, _D + _L), lambda i: (i, 0)),
        out_shape=jax.ShapeDtypeStruct((_N, _D + _L), f32),
    )(features, wp)

    hid0, ea0 = _k1(ftbl, mpT0)
    hid1, ea1 = _k1(ftbl, mpT1)

    U, dn = _k2(hid0, ea0, dst0, hid1, ea1, dst1)

    colsum = pl.pallas_call(
        _t1_body,
        grid=(2, _N // _BN),
        in_specs=[
            pl.BlockSpec((1, _H, _BN, _D), lambda m, i: (m, 0, i, 0)),
            pl.BlockSpec((1, _H, _BN, _L), lambda m, i: (m, 0, i, 0)),
            pl.BlockSpec((_H, _D, _AV), lambda m, i: (0, 0, 0)),
            pl.BlockSpec((1, _AV), lambda m, i: (0, 0)),
        ],
        out_specs=pl.BlockSpec((1, 8, _AV), lambda m, i: (m, 0, 0)),
        out_shape=jax.ShapeDtypeStruct((2, 8, _AV), f32),
    )(U, dn, w1r, fc1_b.reshape(1, _AV))

    s_mean = colsum[:, 0, :] / _N
    beta = jax.nn.softmax((s_mean @ fc2_w.T)[:, 0])

    h_hnd, h_fc = pl.pallas_call(
        _t2_body,
        grid=(_N // _BN,),
        in_specs=[
            pl.BlockSpec((2, _H, _BN, _D), lambda i: (0, 0, i, 0)),
            pl.BlockSpec((2, _H, _BN, _L), lambda i: (0, 0, i, 0)),
            pl.BlockSpec((_BN, 1), lambda i: (i, 0)),
            pl.BlockSpec(memory_space=pltpu.SMEM),
            pl.BlockSpec((_H, _D, _OUT), lambda i: (0, 0, 0)),
            pl.BlockSpec((1, _OUT), lambda i: (0, 0)),
        ],
        out_specs=[
            pl.BlockSpec((_H, _BN, _D), lambda i: (0, i, 0)),
            pl.BlockSpec((_BN, _OUT), lambda i: (i, 0)),
        ],
        out_shape=[
            jax.ShapeDtypeStruct((_H, _N, _D), f32),
            jax.ShapeDtypeStruct((_N, _OUT), f32),
        ],
    )(U, dn, type_mask.reshape(_N, 1), beta, w2r, fc_b.reshape(1, _OUT))

    h = h_hnd.transpose(1, 0, 2).reshape(_N, _H * _D)
    return h_fc, h, beta[:, None, None]


# speedup vs baseline: 9.7592x; 1.0487x over previous
import jax
import jax.numpy as jnp
from jax import lax
from jax.experimental import pallas as pl
from jax.experimental.pallas import tpu as pltpu
from jax.experimental.pallas import tpu_sc as plsc

_N = 10000
_E = 160000
_D = 128
_H = 4
_AV = 128
_OUT = 128

_NC = 2
_NS = 16
_NW = _NC * _NS
_L = 16

_EPT1 = _E // _NW
_B1 = 40
_NB1 = _EPT1 // _B1

_EPT2 = _E // _NS
_B2 = 40
_NB2 = _EPT2 // _B2

_RPS = _N // _NS
_ZR = 100
_ZCH = [(o, min(_ZR, _RPS - o)) for o in range(0, _RPS, _ZR)]

_mesh = plsc.VectorSubcoreMesh(
    core_axis_name="c", subcore_axis_name="s",
    num_cores=_NC, num_subcores=_NS)


def _k1_body(ftbl, mpT0, mpT1, hid0_o, ea0_o, hid1_o, ea1_o,
             rA0, rB0, rC0, hB0, eB0, ix0,
             rA1, rB1, rC1, hB1, eB1, ix1,
             sem0, sem1, semw0, semw1):
    c = lax.axis_index("c")
    s = lax.axis_index("s")
    wid = s * _NC + c
    tbase = wid * _EPT1
    third = jnp.float32(1.0 / 3.0)

    sets = ((rA0, rB0, rC0, hB0, eB0, ix0, sem0, semw0),
            (rA1, rB1, rC1, hB1, eB1, ix1, sem1, semw1))

    for m in range(2):
        mpT = (mpT0, mpT1)[m]
        hid_out = (hid0_o, hid1_o)[m]
        ea_out = (ea0_o, ea1_o)[m]

        def fire(P, j):
            rA, rB, rC, hB, eB, ix, sem, semw = sets[P]
            base = tbase + j * _B1
            pltpu.sync_copy(mpT.at[:, pl.ds(base, _B1)], ix)
            pltpu.async_copy(ftbl.at[ix.at[0]], rA, sem)
            pltpu.async_copy(ftbl.at[ix.at[1]], rB, sem)
            pltpu.async_copy(ftbl.at[ix.at[2]], rC, sem)

        def waitset(P):
            rA, rB, rC, hB, eB, ix, sem, semw = sets[P]
            pltpu.make_async_copy(ftbl.at[ix.at[0]], rA, sem).wait()
            pltpu.make_async_copy(ftbl.at[ix.at[1]], rB, sem).wait()
            pltpu.make_async_copy(ftbl.at[ix.at[2]], rC, sem).wait()

        def wait_w(P):
            rA, rB, rC, hB, eB, ix, sem, semw = sets[P]
            pltpu.make_async_copy(hB, hid_out.at[pl.ds(tbase, _B1)],
                                  semw).wait()
            pltpu.make_async_copy(eB, ea_out.at[pl.ds(tbase, _B1)],
                                  semw).wait()

        def compute(P, j, wait_prev):
            rA, rB, rC, hB, eB, ix, sem, semw = sets[P]
            base = tbase + j * _B1

            @pl.when(wait_prev)
            def _():
                wait_w(P)

            @plsc.parallel_loop(0, _B1, step=1, unroll=4)
            def _(b):
                for q in range(_D // _L):
                    sl = pl.ds(q * _L, _L)
                    hB[b, sl] = (rA[b, sl] + rB[b, sl] + rC[b, sl]) * third
                pp = pl.ds(_D, _L)
                a = (rA[b, pp] + rB[b, pp] + rC[b, pp]) * third
                a = jnp.where(a > 0, a, a * jnp.float32(0.01))
                eB[b, pl.ds(0, _L)] = jnp.exp(a)

            pltpu.async_copy(hB, hid_out.at[pl.ds(base, _B1)], semw)
            pltpu.async_copy(eB, ea_out.at[pl.ds(base, _B1)], semw)

        fire(0, 0)

        def pair(k, _):
            fire(1, 2 * k + 1)
            waitset(0)
            compute(0, 2 * k, k > 0)
            fire(0, 2 * k + 2)
            waitset(1)
            compute(1, 2 * k + 1, k > 0)
            return 0

        lax.fori_loop(0, (_NB1 - 1) // 2, pair, 0)
        waitset(0)
        compute(0, _NB1 - 1, True)
        wait_w(0)
        wait_w(1)


_k1 = pl.kernel(
    _k1_body,
    out_type=(jax.ShapeDtypeStruct((_E, _D), jnp.float32),
              jax.ShapeDtypeStruct((_E, _L), jnp.float32),
              jax.ShapeDtypeStruct((_E, _D), jnp.float32),
              jax.ShapeDtypeStruct((_E, _L), jnp.float32)),
    mesh=_mesh,
    compiler_params=pltpu.CompilerParams(use_tc_tiling_on_sc=False,
                                         needs_layout_passes=False),
    scratch_types=[
        pltpu.VMEM((_B1, _D + _L), jnp.float32),
        pltpu.VMEM((_B1, _D + _L), jnp.float32),
        pltpu.VMEM((_B1, _D + _L), jnp.float32),
        pltpu.VMEM((_B1, _D), jnp.float32),
        pltpu.VMEM((_B1, _L), jnp.float32),
        pltpu.VMEM((3, _B1), jnp.int32),
        pltpu.VMEM((_B1, _D + _L), jnp.float32),
        pltpu.VMEM((_B1, _D + _L), jnp.float32),
        pltpu.VMEM((_B1, _D + _L), jnp.float32),
        pltpu.VMEM((_B1, _D), jnp.float32),
        pltpu.VMEM((_B1, _L), jnp.float32),
        pltpu.VMEM((3, _B1), jnp.int32),
        pltpu.SemaphoreType.DMA,
        pltpu.SemaphoreType.DMA,
        pltpu.SemaphoreType.DMA,
        pltpu.SemaphoreType.DMA,
    ],
)


def _k2_body(hid0, ea0, ei0, hid1, ea1, ei1, U_out, dn_out,
             spU, spD,
             hb0, eb0, mb0, db0, ib0, is0,
             hb1, eb1, mb1, db1, ib1, is1,
             zb, zd, sin0, ssc0, sin1, ssc1):
    c = lax.axis_index("c")
    s = lax.axis_index("s")
    zvec = jnp.zeros((_L,), jnp.float32)

    def z1(i, _):
        zb[i // (_D // _L), pl.ds((i % (_D // _L)) * _L, _L)] = zvec
        return 0
    lax.fori_loop(0, _ZR * (_D // _L), z1, 0)

    def z2(i, _):
        zd[i, pl.ds(0, _L)] = zvec
        return 0
    lax.fori_loop(0, _ZR, z2, 0)

    sets = ((hb0, eb0, mb0, db0, ib0, is0, sin0, ssc0),
            (hb1, eb1, mb1, db1, ib1, is1, sin1, ssc1))
    rbase = s * _RPS
    for m in range(2):
        hid, ea, ei = ((hid0, ea0, ei0), (hid1, ea1, ei1))[m]
        for p in range(2):
            for off, nr in _ZCH:
                pltpu.async_copy(zb.at[pl.ds(0, nr)],
                                 spU.at[pl.ds(rbase + off, nr)], sin0)
                pltpu.async_copy(zd.at[pl.ds(0, nr)],
                                 spD.at[pl.ds(rbase + off, nr)], sin0)
            for off, nr in _ZCH:
                pltpu.make_async_copy(zb.at[pl.ds(0, nr)],
                                      spU.at[pl.ds(rbase + off, nr)],
                                      sin0).wait()
                pltpu.make_async_copy(zd.at[pl.ds(0, nr)],
                                      spD.at[pl.ds(rbase + off, nr)],
                                      sin0).wait()
            plsc.subcore_barrier()

            lane = jnp.int32(4 * m + 2 * p) + c
            tbase = s * _EPT2

            def fire_in(P, j):
                hb, eb, mb, db, ib, isc, sin, ssc = sets[P]
                base = tbase + j * _B2
                pltpu.async_copy(hid.at[pl.ds(base, _B2)], hb, sin)
                pltpu.async_copy(ea.at[pl.ds(base, _B2)], eb, sin)
                pltpu.async_copy(ei.at[1, pl.ds(base, _B2)], ib, sin)

            def wait_in(P):
                hb, eb, mb, db, ib, isc, sin, ssc = sets[P]
                pltpu.make_async_copy(hid.at[pl.ds(tbase, _B2)], hb,
                                      sin).wait()
                pltpu.make_async_copy(ea.at[pl.ds(tbase, _B2)], eb,
                                      sin).wait()
                pltpu.make_async_copy(ei.at[1, pl.ds(tbase, _B2)], ib,
                                      sin).wait()

            def wait_sc(P):
                hb, eb, mb, db, ib, isc, sin, ssc = sets[P]
                pltpu.make_async_copy(mb, spU.at[isc], ssc).wait()
                pltpu.make_async_copy(db, spD.at[isc], ssc).wait()

            def compute_fire(P):
                hb, eb, mb, db, ib, isc, sin, ssc = sets[P]
                starts = list(range(0, _B2 - _L + 1, _L))
                if starts[-1] != _B2 - _L:
                    starts.append(_B2 - _L)
                for st in starts:
                    sl = pl.ds(st, _L)
                    isc[sl] = ib[sl]

                @plsc.parallel_loop(0, _B2, step=1, unroll=4)
                def _(b):
                    wv = plsc.load_gather(
                        eb, (jnp.full((_L,), b, jnp.int32),
                             jnp.full((_L,), lane, jnp.int32)))
                    for q in range(_D // _L):
                        sl = pl.ds(q * _L, _L)
                        mb[b, sl] = hb[b, sl] * wv
                    db[b, pl.ds(0, _L)] = wv

                pltpu.async_copy(mb, spU.at[isc], ssc, add=True)
                pltpu.async_copy(db, spD.at[isc], ssc, add=True)

            fire_in(0, 0)

            def blkpair(k, _):
                fire_in(1, 2 * k + 1)
                wait_in(0)

                @pl.when(k > 0)
                def _():
                    wait_sc(0)
                compute_fire(0)

                @pl.when(k < _NB2 // 2 - 1)
                def _():
                    fire_in(0, 2 * k + 2)
                wait_in(1)

                @pl.when(k > 0)
                def _():
                    wait_sc(1)
                compute_fire(1)
                return 0

            lax.fori_loop(0, _NB2 // 2, blkpair, 0)
            wait_sc(0)
            wait_sc(1)
            plsc.subcore_barrier()

            h0 = 2 * p

            @pl.when(c == 0)
            def _():
                pltpu.sync_copy(spU.at[pl.ds(rbase, _RPS)],
                                U_out.at[m, h0, pl.ds(rbase, _RPS)])
                pltpu.sync_copy(spD.at[pl.ds(rbase, _RPS)],
                                dn_out.at[m, h0, pl.ds(rbase, _RPS)])

            @pl.when(c == 1)
            def _():
                pltpu.sync_copy(spU.at[pl.ds(rbase, _RPS)],
                                U_out.at[m, h0 + 1, pl.ds(rbase, _RPS)])
                pltpu.sync_copy(spD.at[pl.ds(rbase, _RPS)],
                                dn_out.at[m, h0 + 1, pl.ds(rbase, _RPS)])


_k2 = pl.kernel(
    _k2_body,
    out_type=(jax.ShapeDtypeStruct((2, _H, _N, _D), jnp.float32),
              jax.ShapeDtypeStruct((2, _H, _N, _L), jnp.float32)),
    mesh=_mesh,
    compiler_params=pltpu.CompilerParams(use_tc_tiling_on_sc=False,
                                         needs_layout_passes=False),
    scratch_types=[
        pltpu.VMEM_SHARED((_N, _D), jnp.float32),
        pltpu.VMEM_SHARED((_N, _L), jnp.float32),
        pltpu.VMEM((_B2, _D), jnp.float32),
        pltpu.VMEM((_B2, _L), jnp.float32),
        pltpu.VMEM((_B2, _D), jnp.float32),
        pltpu.VMEM((_B2, _L), jnp.float32),
        pltpu.VMEM((_B2,), jnp.int32),
        pltpu.VMEM((_B2,), jnp.int32),
        pltpu.VMEM((_B2, _D), jnp.float32),
        pltpu.VMEM((_B2, _L), jnp.float32),
        pltpu.VMEM((_B2, _D), jnp.float32),
        pltpu.VMEM((_B2, _L), jnp.float32),
        pltpu.VMEM((_B2,), jnp.int32),
        pltpu.VMEM((_B2,), jnp.int32),
        pltpu.VMEM((_ZR, _D), jnp.float32),
        pltpu.VMEM((_ZR, _L), jnp.float32),
        pltpu.SemaphoreType.DMA,
        pltpu.SemaphoreType.DMA,
        pltpu.SemaphoreType.DMA,
        pltpu.SemaphoreType.DMA,
    ],
)


_BN = 1000


def _t0_body(f_ref, w_ref, o_ref):
    f = f_ref[...]
    o_ref[:, 0:_D] = f
    o_ref[:, _D:_D + _L] = jnp.dot(f, w_ref[...],
                                   preferred_element_type=jnp.float32)


def _elu_rows(u_ref, d_ref, m, h):
    u = u_ref[m, h]
    d = d_ref[m, h, :, 0:1]
    r = jnp.where(d > 0, u / d, 0.0)
    return jnp.where(r > 0, r, jnp.exp(r) - 1.0)


def _t1_body(u_ref, d_ref, w_ref, b_ref, o_ref):
    i = pl.program_id(1)

    @pl.when(i == 0)
    def _():
        o_ref[...] = jnp.zeros_like(o_ref)

    acc = jnp.broadcast_to(b_ref[...], (_BN, _AV))
    for h in range(_H):
        o = _elu_rows(u_ref, d_ref, 0, h)
        acc = acc + jnp.dot(o, w_ref[h], preferred_element_type=jnp.float32)
    col = jnp.sum(jnp.tanh(acc), axis=0, keepdims=True)
    o_ref[...] += jnp.broadcast_to(col, (1, 8, _AV))


def _t2_body(u_ref, d_ref, tm_ref, beta_ref, w2_ref, b2_ref, h_ref, hfc_ref):
    mask = tm_ref[...] == 0
    b0 = beta_ref[0]
    b1 = beta_ref[1]
    acc = jnp.broadcast_to(b2_ref[...], (_BN, _OUT))
    for h in range(_H):
        hb = b0 * _elu_rows(u_ref, d_ref, 0, h) + b1 * _elu_rows(u_ref, d_ref, 1, h)
        hb = jnp.where(mask, hb, 0.0)
        h_ref[h] = hb
        acc = acc + jnp.dot(hb, w2_ref[h], preferred_element_type=jnp.float32)
    hfc_ref[...] = acc


def kernel(features, type_mask, edge_index0, mp_idx0, edge_index1, mp_idx1,
           attn0, attn1, fc1_w, fc1_b, fc2_w, fc_w, fc_b):
    f32 = jnp.float32
    wp = jnp.concatenate(
        [attn0[0].T, attn1[0].T, jnp.zeros((_D, 2 * _H), f32)], axis=1)
    mpT0 = mp_idx0.T
    mpT1 = mp_idx1.T
    w1r = fc1_w.reshape(_AV, _H, _D).transpose(1, 2, 0)
    w2r = fc_w.reshape(_OUT, _H, _D).transpose(1, 2, 0)

    ftbl = pl.pallas_call(
        _t0_body,
        grid=(_N // _BN,),
        in_specs=[pl.BlockSpec((_BN, _D), lambda i: (i, 0)),
                  pl.BlockSpec((_D, _L), lambda i: (0, 0))],
        out_specs=pl.BlockSpec((_BN, _D + _L), lambda i: (i, 0)),
        out_shape=jax.ShapeDtypeStruct((_N, _D + _L), f32),
    )(features, wp)

    hid0, ea0, hid1, ea1 = _k1(ftbl, mpT0, mpT1)

    U, dn = _k2(hid0, ea0, edge_index0, hid1, ea1, edge_index1)

    colsum = pl.pallas_call(
        _t1_body,
        grid=(2, _N // _BN),
        in_specs=[
            pl.BlockSpec((1, _H, _BN, _D), lambda m, i: (m, 0, i, 0)),
            pl.BlockSpec((1, _H, _BN, _L), lambda m, i: (m, 0, i, 0)),
            pl.BlockSpec((_H, _D, _AV), lambda m, i: (0, 0, 0)),
            pl.BlockSpec((1, _AV), lambda m, i: (0, 0)),
        ],
        out_specs=pl.BlockSpec((1, 8, _AV), lambda m, i: (m, 0, 0)),
        out_shape=jax.ShapeDtypeStruct((2, 8, _AV), f32),
    )(U, dn, w1r, fc1_b.reshape(1, _AV))

    s_mean = colsum[:, 0, :] / _N
    beta = jax.nn.softmax((s_mean @ fc2_w.T)[:, 0])

    h_hnd, h_fc = pl.pallas_call(
        _t2_body,
        grid=(_N // _BN,),
        in_specs=[
            pl.BlockSpec((2, _H, _BN, _D), lambda i: (0, 0, i, 0)),
            pl.BlockSpec((2, _H, _BN, _L), lambda i: (0, 0, i, 0)),
            pl.BlockSpec((_BN, 1), lambda i: (i, 0)),
            pl.BlockSpec(memory_space=pltpu.SMEM),
            pl.BlockSpec((_H, _D, _OUT), lambda i: (0, 0, 0)),
            pl.BlockSpec((1, _OUT), lambda i: (0, 0)),
        ],
        out_specs=[
            pl.BlockSpec((_H, _BN, _D), lambda i: (0, i, 0)),
            pl.BlockSpec((_BN, _OUT), lambda i: (i, 0)),
        ],
        out_shape=[
            jax.ShapeDtypeStruct((_H, _N, _D), f32),
            jax.ShapeDtypeStruct((_N, _OUT), f32),
        ],
    )(U, dn, type_mask.reshape(_N, 1), beta, w2r, fc_b.reshape(1, _OUT))

    h = h_hnd.transpose(1, 0, 2).reshape(_N, _H * _D)
    return h_fc, h, beta[:, None, None]
